# Initial kernel scaffold; baseline (speedup 1.0000x reference)
import jax
import jax.numpy as jnp
from jax.experimental import pallas as pl


def kernel(x, ln_gamma, ln_beta, W_proj, b_proj, sim, temperature, W_down, b_down, W_up, b_up):
    raise NotImplementedError("write your pallas kernel here")


# speedup vs baseline: 2.3791x; 2.3791x over previous
import functools

import jax
import jax.numpy as jnp
from jax import lax
from jax.experimental import pallas as pl

_B, _S, _D = 4, 2048, 1024
_E = 8
_BOT = 64
_PROJ = 256
_N = _B * _S
_TN = 512
_EPS = 1e-12

_HI = jax.lax.Precision.HIGHEST


def _fused_body(x_ref, g_ref, be_ref, wp_ref, bp_ref, sim_ref, temp_ref,
                wd_ref, bd_ref, wu_ref, bup_ref, out_ref):
    xb = x_ref[...]
    mean = jnp.mean(xb, axis=1, keepdims=True)
    xc = xb - mean
    var = jnp.mean(xc * xc, axis=1, keepdims=True)
    xn = xc / jnp.sqrt(var + 1e-5) * g_ref[...] + be_ref[...]

    proj = jnp.dot(xn, wp_ref[...],
                   preferred_element_type=jnp.float32) + bp_ref[...]
    pnorm = jnp.sqrt(jnp.sum(proj * proj, axis=1, keepdims=True))
    proj = proj / jnp.maximum(pnorm, _EPS)
    sim = sim_ref[...]
    snorm = jnp.sqrt(jnp.sum(sim * sim, axis=0, keepdims=True))
    simn = sim / jnp.maximum(snorm, _EPS)
    scale = jnp.exp(jnp.minimum(temp_ref[0, 0], jnp.log(jnp.float32(100.0))))
    logits = jnp.dot(proj, simn,
                     preferred_element_type=jnp.float32) * scale

    ii = lax.broadcasted_iota(jnp.int32, (_TN, _E), 1)
    m1 = jnp.max(logits, axis=1, keepdims=True)
    idx1 = jnp.min(jnp.where(logits == m1, ii, _E), axis=1, keepdims=True)
    oh1 = ii == idx1
    logits2 = jnp.where(oh1, -jnp.inf, logits)
    m2 = jnp.max(logits2, axis=1, keepdims=True)
    idx2 = jnp.min(jnp.where(logits2 == m2, ii, _E), axis=1, keepdims=True)
    oh2 = ii == idx2
    w2 = 1.0 / (1.0 + jnp.exp(m1 - m2))
    w1 = 1.0 - w2
    weights = jnp.where(oh1, w1, 0.0) + jnp.where(oh2, w2, 0.0)

    h = jnp.dot(xn, wd_ref[...], preferred_element_type=jnp.float32)
    h = jnp.maximum(h + bd_ref[...], 0.0)
    jj = lax.broadcasted_iota(jnp.int32, (_E, _E * _BOT), 1) // _BOT
    ee = lax.broadcasted_iota(jnp.int32, (_E, _E * _BOT), 0)
    expand = jnp.where(jj == ee, 1.0, 0.0).astype(jnp.float32)
    wexp = jnp.dot(weights, expand, preferred_element_type=jnp.float32,
                   precision=_HI)
    up = jnp.dot(h * wexp, wu_ref[...], preferred_element_type=jnp.float32)
    bup = jnp.dot(weights, bup_ref[...], preferred_element_type=jnp.float32,
                  precision=_HI)
    out_ref[...] = xn + up + bup


@jax.jit
def _fused(x_flat, g2, be2, W_proj, bp2, sim, temp2, Wd_flat, bd2, Wu_flat, b_up):
    grid = (_N // _TN,)
    return pl.pallas_call(
        _fused_body,
        grid=grid,
        in_specs=[
            pl.BlockSpec((_TN, _D), lambda i: (i, 0)),
            pl.BlockSpec((1, _D), lambda i: (0, 0)),
            pl.BlockSpec((1, _D), lambda i: (0, 0)),
            pl.BlockSpec((_D, _PROJ), lambda i: (0, 0)),
            pl.BlockSpec((1, _PROJ), lambda i: (0, 0)),
            pl.BlockSpec((_PROJ, _E), lambda i: (0, 0)),
            pl.BlockSpec((1, 1), lambda i: (0, 0)),
            pl.BlockSpec((_D, _E * _BOT), lambda i: (0, 0)),
            pl.BlockSpec((1, _E * _BOT), lambda i: (0, 0)),
            pl.BlockSpec((_E * _BOT, _D), lambda i: (0, 0)),
            pl.BlockSpec((_E, _D), lambda i: (0, 0)),
        ],
        out_specs=pl.BlockSpec((_TN, _D), lambda i: (i, 0)),
        out_shape=jax.ShapeDtypeStruct((_N, _D), jnp.float32),
    )(x_flat, g2, be2, W_proj, bp2, sim, temp2, Wd_flat, bd2, Wu_flat, b_up)


def kernel(x, ln_gamma, ln_beta, W_proj, b_proj, sim, temperature, W_down, b_down, W_up, b_up):
    x_flat = x.reshape(_N, _D)
    Wd_flat = W_down.transpose(1, 0, 2).reshape(_D, _E * _BOT)
    Wu_flat = W_up.reshape(_E * _BOT, _D)
    out = _fused(
        x_flat,
        ln_gamma.reshape(1, _D),
        ln_beta.reshape(1, _D),
        W_proj,
        b_proj.reshape(1, _PROJ),
        sim,
        temperature.reshape(1, 1),
        Wd_flat,
        b_down.reshape(1, _E * _BOT),
        Wu_flat,
        b_up,
    )
    return out.reshape(_B, _S, _D), jnp.asarray(0.0, jnp.float32)
